# baseline (speedup 1.0000x reference)
import functools

import jax
import jax.numpy as jnp
from jax import lax
from jax.experimental import pallas as pl
from jax.experimental.pallas import tpu as pltpu
from jax.experimental.pallas import tpu_sc as plsc

_N = 640000
_D = 128
_NW = 32
_BT = 5120
_M = 327680
_NSC = _N - _M
_BLK = 2560
_CH = _D
_NCHB = _BLK // _CH
_NBLK = _NSC // _BLK
_BLK_PER_W = -(-_NBLK // _NW)


def _combo_body(w0_ref, w1_ref, w2_ref, combo_ref):
    c01 = (w0_ref[...][:, None, :] + w1_ref[...][None, :, :]).reshape(30, _D)
    combo_ref[...] = (c01[:, None, :] + w2_ref[...][None, :, :]).reshape(60, _D)


def _build_combo(W0, W1, W2):
    return pl.pallas_call(
        _combo_body,
        out_shape=jax.ShapeDtypeStruct((60, _D), jnp.float32),
    )(W0, W1, W2)


def _tc_body(e0_ref, e1_ref, e2_ref, w0_ref, w1_ref, w2_ref, out_ref):
    acc = None
    for e_ref, w_ref, v in ((e0_ref, w0_ref, 5), (e1_ref, w1_ref, 6), (e2_ref, w2_ref, 2)):
        idx = e_ref[0]
        oh = (jax.lax.broadcasted_iota(jnp.int32, (v, _BT), 0) == idx).astype(jnp.float32)
        part = jax.lax.dot_general(
            oh, w_ref[...],
            dimension_numbers=(((0,), (0,)), ((), ())),
            preferred_element_type=jnp.float32,
        )
        acc = part if acc is None else acc + part
    out_ref[...] = acc


def _tc_lookup(e0, e1, e2, W0, W1, W2):
    nb = _M // _BT
    espec = pl.BlockSpec((1, 1, _BT), lambda i: (i, 0, 0))
    wspec = lambda v: pl.BlockSpec((v, _D), lambda i: (0, 0))
    return pl.pallas_call(
        _tc_body,
        grid=(nb,),
        in_specs=[espec, espec, espec, wspec(5), wspec(6), wspec(2)],
        out_specs=pl.BlockSpec((_BT, _D), lambda i: (i, 0)),
        out_shape=jax.ShapeDtypeStruct((_M, _D), jnp.float32),
    )(e0.reshape(nb, 1, _BT), e1.reshape(nb, 1, _BT), e2.reshape(nb, 1, _BT),
      W0, W1, W2)


@functools.partial(
    pl.kernel,
    out_type=jax.ShapeDtypeStruct((_NSC, _D), jnp.float32),
    mesh=plsc.VectorSubcoreMesh(core_axis_name="c", subcore_axis_name="s"),
    scratch_types=[
        pltpu.VMEM((_BLK,), jnp.int32),
        pltpu.VMEM((_BLK,), jnp.int32),
        pltpu.VMEM((_BLK,), jnp.int32),
        pltpu.VMEM((_BLK // _D, _D), jnp.int32),
        pltpu.VMEM((_D, _D), jnp.float32),
        pltpu.VMEM((_D, _D), jnp.float32),
        pltpu.VMEM_SHARED((60, _D), jnp.float32),
        pltpu.SemaphoreType.DMA,
        pltpu.SemaphoreType.DMA,
        pltpu.SemaphoreType.DMA,
        pltpu.SemaphoreType.DMA,
    ],
)
def _sc_lookup(e0, e1, e2, combo, out, e0_v, e1_v, e2_v, c_v,
               rows_a, rows_b, combo_sh, sem_ga, sem_gb, sem_oa, sem_ob):
    sid = lax.axis_index("s")
    wid = sid * 2 + lax.axis_index("c")
    rows_bufs = (rows_a, rows_b)
    sems_g = (sem_ga, sem_gb)
    sems_o = (sem_oa, sem_ob)

    @pl.when(sid == 0)
    def _stage():
        pltpu.sync_copy(combo, combo_sh)

    plsc.subcore_barrier()

    def gather_start(j, buf):
        return pltpu.async_copy(
            combo_sh.at[c_v.at[j]], rows_bufs[buf], sems_g[buf])

    def out_start(bid, j, buf):
        row = bid * _BLK + j * _CH
        return pltpu.async_copy(
            rows_bufs[buf], out.at[pl.ds(row, _CH)], sems_o[buf])

    def blk_body(ib, carry):
        bid = ib * _NW + wid

        @pl.when(bid < _NBLK)
        def _run():
            base_b = bid * _BLK
            pltpu.sync_copy(e0.at[pl.ds(base_b, _BLK)], e0_v)
            pltpu.sync_copy(e1.at[pl.ds(base_b, _BLK)], e1_v)
            pltpu.sync_copy(e2.at[pl.ds(base_b, _BLK)], e2_v)

            def c_body(i, c2):
                s = pl.ds(i * 16, 16)
                c_v[i // 8, pl.ds((i % 8) * 16, 16)] = (
                    e0_v[s] * 12 + e1_v[s] * 2 + e2_v[s])
                return c2

            lax.fori_loop(0, _BLK // 16, c_body, 0)

            gd = [None, None]
            od = [None, None]
            gd[0] = gather_start(0, 0)
            for j in range(1, _NCHB):
                buf, prev = j % 2, (j - 1) % 2
                if j >= 2:
                    od[buf].wait()
                gd[buf] = gather_start(j, buf)
                gd[prev].wait()
                od[prev] = out_start(bid, j - 1, prev)
            last = (_NCHB - 1) % 2
            gd[last].wait()
            od[last] = out_start(bid, _NCHB - 1, last)
            od[1 - last].wait()
            od[last].wait()

        return carry

    lax.fori_loop(0, _BLK_PER_W, blk_body, 0)


def kernel(edge_attr, W0, W1, W2):
    ea = edge_attr.astype(jnp.int32)
    e0, e1, e2 = ea[:, 0], ea[:, 1], ea[:, 2]
    combo = _build_combo(W0, W1, W2)
    sc_out = _sc_lookup(e0[_M:], e1[_M:], e2[_M:], combo)
    tc_out = _tc_lookup(e0[:_M], e1[:_M], e2[:_M], W0, W1, W2)
    return jnp.concatenate([tc_out, sc_out], axis=0)


# speedup vs baseline: 2.0686x; 2.0686x over previous
import functools

import jax
import jax.numpy as jnp
from jax import lax
from jax.experimental import pallas as pl
from jax.experimental.pallas import tpu as pltpu
from jax.experimental.pallas import tpu_sc as plsc

_N = 640000
_D = 128
_NW = 32
_BLK = 5120
_CH = _D
_NCHB = _BLK // _CH
_NBLK = _N // _BLK
_BLK_PER_W = -(-_NBLK // _NW)


def _combo_body(w0_ref, w1_ref, w2_ref, combo_ref):
    c01 = (w0_ref[...][:, None, :] + w1_ref[...][None, :, :]).reshape(30, _D)
    combo_ref[...] = (c01[:, None, :] + w2_ref[...][None, :, :]).reshape(60, _D)


def _build_combo(W0, W1, W2):
    return pl.pallas_call(
        _combo_body,
        out_shape=jax.ShapeDtypeStruct((60, _D), jnp.float32),
    )(W0, W1, W2)


@functools.partial(
    pl.kernel,
    out_type=jax.ShapeDtypeStruct((_N, _D), jnp.float32),
    mesh=plsc.VectorSubcoreMesh(core_axis_name="c", subcore_axis_name="s"),
    scratch_types=[
        pltpu.VMEM((_BLK,), jnp.int32),
        pltpu.VMEM((_BLK,), jnp.int32),
        pltpu.VMEM((_BLK,), jnp.int32),
        pltpu.VMEM((_BLK // _D, _D), jnp.int32),
        pltpu.VMEM((_D, _D), jnp.float32),
        pltpu.VMEM((_D, _D), jnp.float32),
        pltpu.VMEM_SHARED((60, _D), jnp.float32),
        pltpu.SemaphoreType.DMA,
        pltpu.SemaphoreType.DMA,
        pltpu.SemaphoreType.DMA,
        pltpu.SemaphoreType.DMA,
    ],
)
def _sc_lookup(e0, e1, e2, combo, out, e0_v, e1_v, e2_v, c_v,
               rows_a, rows_b, combo_sh, sem_ga, sem_gb, sem_oa, sem_ob):
    sid = lax.axis_index("s")
    wid = sid * 2 + lax.axis_index("c")
    rows_bufs = (rows_a, rows_b)
    sems_g = (sem_ga, sem_gb)
    sems_o = (sem_oa, sem_ob)

    @pl.when(sid == 0)
    def _stage():
        pltpu.sync_copy(combo, combo_sh)

    plsc.subcore_barrier()

    def gather_start(j, buf):
        return pltpu.async_copy(
            combo_sh.at[c_v.at[j]], rows_bufs[buf], sems_g[buf])

    def out_start(bid, j, buf):
        row = bid * _BLK + j * _CH
        return pltpu.async_copy(
            rows_bufs[buf], out.at[pl.ds(row, _CH)], sems_o[buf])

    def blk_body(ib, carry):
        bid = ib * _NW + wid

        @pl.when(bid < _NBLK)
        def _run():
            base_b = bid * _BLK
            pltpu.sync_copy(e0.at[pl.ds(base_b, _BLK)], e0_v)
            pltpu.sync_copy(e1.at[pl.ds(base_b, _BLK)], e1_v)
            pltpu.sync_copy(e2.at[pl.ds(base_b, _BLK)], e2_v)

            def c_body(i, c2):
                s = pl.ds(i * 16, 16)
                c_v[i // 8, pl.ds((i % 8) * 16, 16)] = (
                    e0_v[s] * 12 + e1_v[s] * 2 + e2_v[s])
                return c2

            lax.fori_loop(0, _BLK // 16, c_body, 0)

            gd = [None, None]
            od = [None, None]
            gd[0] = gather_start(0, 0)
            for j in range(1, _NCHB):
                buf, prev = j % 2, (j - 1) % 2
                if j >= 2:
                    od[buf].wait()
                gd[buf] = gather_start(j, buf)
                gd[prev].wait()
                od[prev] = out_start(bid, j - 1, prev)
            last = (_NCHB - 1) % 2
            gd[last].wait()
            od[last] = out_start(bid, _NCHB - 1, last)
            od[1 - last].wait()
            od[last].wait()

        return carry

    lax.fori_loop(0, _BLK_PER_W, blk_body, 0)


def kernel(edge_attr, W0, W1, W2):
    ea = edge_attr.astype(jnp.int32)
    combo = _build_combo(W0, W1, W2)
    return _sc_lookup(ea[:, 0], ea[:, 1], ea[:, 2], combo)
